# baseline (speedup 1.0000x reference)
import jax
import jax.numpy as jnp
from jax.experimental import pallas as pl
from jax.experimental.pallas import tpu as pltpu

_SEG = 8


def _pos_body(col_ref, row_ref, out_ref, scratch, sem):
    w, d = col_ref.shape
    h = row_ref.shape[0]
    b = out_ref.shape[0]
    rs = h // _SEG
    col = col_ref[...]
    copies = []
    for s in range(_SEG):
        row_s = row_ref[pl.ds(s * rs, rs), :]
        left = jnp.broadcast_to(col[None, :, :], (rs, w, d)).reshape(rs * w, d)
        right = jnp.broadcast_to(row_s[:, None, :], (rs, w, d)).reshape(rs * w, d)
        lo = s * rs * w
        scratch[pl.ds(lo, rs * w), 0:d] = left
        scratch[pl.ds(lo, rs * w), d:2 * d] = right
        for i in range(b):
            cp = pltpu.make_async_copy(
                scratch.at[pl.ds(lo, rs * w)],
                out_ref.at[i, pl.ds(lo, rs * w), :],
                sem.at[i % 2],
            )
            cp.start()
            copies.append(cp)
    for cp in copies:
        cp.wait()


def kernel(x, row_embed, col_embed):
    b = x.shape[0]
    h, w = x.shape[-3], x.shape[-2]
    d = row_embed.shape[1]
    col = col_embed[:w]
    row = row_embed[:h]
    return pl.pallas_call(
        _pos_body,
        in_specs=[
            pl.BlockSpec((w, d), lambda: (0, 0)),
            pl.BlockSpec((h, d), lambda: (0, 0)),
        ],
        out_specs=pl.BlockSpec(memory_space=pl.ANY),
        out_shape=jax.ShapeDtypeStruct((b, h * w, 2 * d), jnp.float32),
        scratch_shapes=[
            pltpu.VMEM((h * w, 2 * d), jnp.float32),
            pltpu.SemaphoreType.DMA((2,)),
        ],
    )(col, row)


# speedup vs baseline: 1.0063x; 1.0063x over previous
import jax
import jax.numpy as jnp
from jax.experimental import pallas as pl
from jax.experimental.pallas import tpu as pltpu

_SEG = 8


def _pos_body(col_ref, row_ref, out_ref, scratch, sem):
    w, d = col_ref.shape
    h = row_ref.shape[0]
    b = out_ref.shape[0]
    rs = h // _SEG
    col = col_ref[...]
    copies = []
    for s in range(_SEG):
        row_s = row_ref[pl.ds(s * rs, rs), :]
        left = jnp.broadcast_to(col[None, :, :], (rs, w, d)).reshape(rs * w, d)
        right = jnp.broadcast_to(row_s[:, None, :], (rs, w, d)).reshape(rs * w, d)
        lo = s * rs * w
        scratch[pl.ds(lo, rs * w), 0:d] = left
        scratch[pl.ds(lo, rs * w), d:2 * d] = right
        for i in range(b):
            cp = pltpu.make_async_copy(
                scratch.at[pl.ds(lo, rs * w)],
                out_ref.at[i, pl.ds(lo, rs * w), :],
                sem.at[i % 8],
            )
            cp.start()
            copies.append(cp)
    for cp in copies:
        cp.wait()


def kernel(x, row_embed, col_embed):
    b = x.shape[0]
    h, w = x.shape[-3], x.shape[-2]
    d = row_embed.shape[1]
    col = col_embed[:w]
    row = row_embed[:h]
    return pl.pallas_call(
        _pos_body,
        in_specs=[
            pl.BlockSpec((w, d), lambda: (0, 0)),
            pl.BlockSpec((h, d), lambda: (0, 0)),
        ],
        out_specs=pl.BlockSpec(memory_space=pl.ANY),
        out_shape=jax.ShapeDtypeStruct((b, h * w, 2 * d), jnp.float32),
        scratch_shapes=[
            pltpu.VMEM((h * w, 2 * d), jnp.float32),
            pltpu.SemaphoreType.DMA((8,)),
        ],
    )(col, row)
